# baseline (speedup 1.0000x reference)
import dataclasses
import functools

import jax
import jax.numpy as jnp
from jax import lax
from jax.experimental import pallas as pl
from jax.experimental.pallas import tpu as pltpu
from jax.experimental.pallas import tpu_sc as plsc

N = 10000
N_PAD = 10240
D = 128
E_RAW = 320000
LANES = 16
K_CHUNK = 32
NB = 5
NSLOT = 10
GD = 3
ID = 5
NC = 2
NS = 16
N_WORKERS = NC * NS
E_PW = 10240
CHUNKS = E_PW // K_CHUNK
E_PAD = E_PW * N_WORKERS
ROWS_PS = N_PAD // NS
WB_ROWS = 128

ROW_BLK = 1024
C_CURV = 5.0
K_CURV = 1.0 / C_CURV
SQRT_K = K_CURV ** 0.5

_SC_MESH = plsc.VectorSubcoreMesh(core_axis_name="c", subcore_axis_name="s")

_SC_PARAMS = pltpu.CompilerParams()
if "needs_layout_passes" in pltpu.CompilerParams.__dataclass_fields__:
    _SC_PARAMS = dataclasses.replace(_SC_PARAMS, needs_layout_passes=False)


def _make_sc_segment(compute_deg):
    out_type = [jax.ShapeDtypeStruct((NC, N_PAD, D), jnp.float32)]
    scratch = (
        [pltpu.VMEM((K_CHUNK,), jnp.int32) for _ in range(NSLOT)]
        + [pltpu.VMEM((K_CHUNK,), jnp.int32) for _ in range(NSLOT)]
        + [pltpu.VMEM((K_CHUNK,), jnp.float32) for _ in range(NSLOT)]
        + [pltpu.VMEM((K_CHUNK, D), jnp.float32) for _ in range(NB)]
        + [pltpu.VMEM_SHARED((N_PAD, D), jnp.float32)]
        + [pltpu.SemaphoreType.DMA for _ in range(NSLOT)]
        + [pltpu.SemaphoreType.DMA for _ in range(NSLOT)]
        + [pltpu.SemaphoreType.DMA for _ in range(NB)]
        + [pltpu.SemaphoreType.DMA for _ in range(NB)]
    )
    if compute_deg:
        out_type.append(jax.ShapeDtypeStruct((NC, N_PAD), jnp.float32))
        scratch.append(pltpu.VMEM_SHARED((N_PAD,), jnp.float32))
        scratch.append(pltpu.VMEM((N_PAD // NS,), jnp.float32))

    @functools.partial(pl.kernel, out_type=out_type, mesh=_SC_MESH,
                       scratch_types=scratch, compiler_params=_SC_PARAMS)
    def sc_seg(x_hbm, src_hbm, dst_hbm, w_hbm, agg_hbm, *rest):
        if compute_deg:
            deg_hbm = rest[0]
            rest = rest[1:]
        it = iter(rest)
        srcv = [next(it) for _ in range(NSLOT)]
        dstv = [next(it) for _ in range(NSLOT)]
        wv = [next(it) for _ in range(NSLOT)]
        rows = [next(it) for _ in range(NB)]
        acc_sp = next(it)
        sem_is = [next(it) for _ in range(NSLOT)]
        sem_idw = [next(it) for _ in range(NSLOT)]
        sem_g = [next(it) for _ in range(NB)]
        sem_s = [next(it) for _ in range(NB)]
        if compute_deg:
            deg_sp = next(it)
            zdeg = next(it)
        ci = lax.axis_index("c")
        si = lax.axis_index("s")
        ebase = (ci * NS + si) * E_PW
        DEG_PS = N_PAD // NS

        def load_idx(slot, c):
            eb = ebase + c * K_CHUNK
            pltpu.async_copy(src_hbm.at[pl.ds(eb, K_CHUNK)], srcv[slot],
                             sem_is[slot])
            pltpu.async_copy(dst_hbm.at[pl.ds(eb, K_CHUNK)], dstv[slot],
                             sem_idw[slot])
            pltpu.async_copy(w_hbm.at[pl.ds(eb, K_CHUNK)], wv[slot],
                             sem_idw[slot])

        def issue_gather(slot, b):
            pltpu.make_async_copy(src_hbm.at[pl.ds(0, K_CHUNK)], srcv[slot],
                                  sem_is[slot]).wait()
            pltpu.async_copy(x_hbm.at[srcv[slot]], rows[b], sem_g[b])

        def wait_scatter(b, slot):
            pltpu.make_async_copy(rows[b], acc_sp.at[dstv[slot]],
                                  sem_s[b]).wait()
            if compute_deg:
                pltpu.make_async_copy(wv[slot], deg_sp.at[dstv[slot]],
                                      sem_s[b]).wait()

        def process(slot, b):
            pltpu.make_async_copy(x_hbm.at[srcv[slot]], rows[b],
                                  sem_g[b]).wait()
            pltpu.make_async_copy(dst_hbm.at[pl.ds(0, K_CHUNK)], dstv[slot],
                                  sem_idw[slot]).wait()
            pltpu.make_async_copy(w_hbm.at[pl.ds(0, K_CHUNK)], wv[slot],
                                  sem_idw[slot]).wait()

            @pl.loop(0, K_CHUNK, unroll=4)
            def _scale(e):
                wvec = plsc.load_gather(
                    wv[slot], [jnp.zeros((LANES,), jnp.int32) + e])
                for j in range(0, D, LANES):
                    rows[b][e, pl.ds(j, LANES)] = (
                        rows[b][e, pl.ds(j, LANES)] * wvec)

            pltpu.async_copy(rows[b], acc_sp.at[dstv[slot]], sem_s[b],
                             add=True)
            if compute_deg:
                pltpu.async_copy(wv[slot], deg_sp.at[dstv[slot]], sem_s[b],
                                 add=True)

        for c in range(ID):
            load_idx(c, c)

        zbuf = rows[NB - 1]

        @pl.loop(0, K_CHUNK)
        def _zrow(i):
            for j in range(0, D, LANES):
                zbuf[i, pl.ds(j, LANES)] = jnp.zeros((LANES,), jnp.float32)

        @pl.loop(0, ROWS_PS // K_CHUNK)
        def _zacc(k):
            rb = si * ROWS_PS + k * K_CHUNK
            pltpu.sync_copy(zbuf, acc_sp.at[pl.ds(rb, K_CHUNK)])

        if compute_deg:
            @pl.loop(0, DEG_PS, step=LANES)
            def _zdeg(i):
                zdeg[pl.ds(i, LANES)] = jnp.zeros((LANES,), jnp.float32)
            pltpu.sync_copy(zdeg, deg_sp.at[pl.ds(si * DEG_PS, DEG_PS)])

        for c in range(GD):
            issue_gather(c, c)
        plsc.subcore_barrier()

        @pl.loop(0, CHUNKS // NSLOT)
        def _ring(cc):
            base = cc * NSLOT
            for k in range(NSLOT):
                c = base + k
                b = k % NB
                bg = (k + GD) % NB
                slotg = (k + GD) % NSLOT
                slot_prev = (k + NSLOT + GD - NB) % NSLOT
                sloti = (k + ID) % NSLOT

                if k < NB - GD:
                    @pl.when(cc > 0)
                    def _(bg=bg, slot_prev=slot_prev):
                        wait_scatter(bg, slot_prev)
                else:
                    wait_scatter(bg, slot_prev)

                if k < NSLOT - GD:
                    issue_gather(slotg, bg)
                else:
                    @pl.when(cc < CHUNKS // NSLOT - 1)
                    def _(slotg=slotg, bg=bg):
                        issue_gather(slotg, bg)

                if k < NSLOT - ID:
                    load_idx(sloti, c + ID)
                else:
                    @pl.when(cc < CHUNKS // NSLOT - 1)
                    def _(sloti=sloti, c=c):
                        load_idx(sloti, c + ID)

                process(k, b)

        for c in range(CHUNKS - (NB - GD), CHUNKS):
            wait_scatter(c % NB, c % NSLOT)
        plsc.subcore_barrier()

        @pl.loop(0, ROWS_PS // K_CHUNK)
        def _wb(k):
            rb = si * ROWS_PS + k * K_CHUNK
            pltpu.sync_copy(acc_sp.at[pl.ds(rb, K_CHUNK)],
                            agg_hbm.at[ci].at[pl.ds(rb, K_CHUNK)])
        if compute_deg:
            pltpu.sync_copy(deg_sp.at[pl.ds(si * DEG_PS, DEG_PS)],
                            deg_hbm.at[ci].at[pl.ds(si * DEG_PS, DEG_PS)])

    return sc_seg


_sc_seg_first = _make_sc_segment(True)
_sc_seg_rest = _make_sc_segment(False)


def _preprocess_body(u_ref, o_ref):
    u = u_ref[...]
    col = lax.broadcasted_iota(jnp.int32, (1, D), 1)
    sp_mask = (col >= 1).astype(jnp.float32)
    usp = u * sp_mask
    s_sp = jnp.sum(usp * usp, axis=1, keepdims=True)
    xn = jnp.maximum(jnp.sqrt(s_sp), 1e-15)
    theta = xn / SQRT_K
    et = jnp.exp(theta)
    emt = jnp.exp(-theta)
    sinh_t = 0.5 * (et - emt)
    scale1 = SQRT_K * sinh_t / xn
    sp1 = scale1 * usp
    s_sp1 = jnp.sum(sp1 * sp1, axis=1, keepdims=True)
    time2 = jnp.sqrt(jnp.maximum(K_CURV + s_sp1, 1e-15))
    alpha = jnp.maximum(time2, 1.0 + 1e-7)
    sn = jnp.maximum(jnp.sqrt(s_sp1), 1e-15)
    dist = jnp.log(alpha + jnp.sqrt(alpha * alpha - 1.0))
    o_ref[...] = (dist / sn) * sp1


def _edge_w_body(d_ref, w_ref):
    d = d_ref[...]
    w_ref[...] = jnp.exp(-(d * d))


def _layer_body(agg_ref, deginv_ref, xin_ref, w_ref, b_ref, acc_ref,
                x_ref, accout_ref, deginv_out, *, first, final):
    if first:
        deg = deginv_ref[0] + deginv_ref[1] + 1.0
        deginv = (1.0 / jnp.maximum(deg, 1e-9))[:, None]
        deginv_out[...] = deginv
    else:
        deginv = deginv_ref[...]
    a = (agg_ref[0] + agg_ref[1] + xin_ref[...]) * deginv
    h = lax.dot_general(a, w_ref[...], (((1,), (0,)), ((), ())),
                        precision=lax.Precision.HIGHEST,
                        preferred_element_type=jnp.float32)
    h = h + b_ref[...]
    x = jnp.where(h > 0.0, h, 0.01 * h)
    x_ref[...] = x
    acc = acc_ref[...] + x
    if final:
        acc = acc * 0.25
    accout_ref[...] = acc


def kernel(poi_embs, edge_index, edge_attr, W0, b0, W1, b1, W2, b2):
    n = poi_embs.shape[0]
    pad = E_PAD - E_RAW
    pad_idx = jnp.arange(pad, dtype=edge_index.dtype) % n
    src = jnp.concatenate([edge_index[0], pad_idx])
    dst = jnp.concatenate([edge_index[1], pad_idx])
    dist = jnp.concatenate([
        edge_attr,
        jnp.full((pad,), 100.0, dtype=edge_attr.dtype),
    ])

    w = pl.pallas_call(
        _edge_w_body,
        out_shape=jax.ShapeDtypeStruct((E_PAD,), jnp.float32),
    )(dist)

    u_pad = jnp.concatenate(
        [poi_embs, jnp.zeros((N_PAD - N, D), jnp.float32)], axis=0)
    x0 = pl.pallas_call(
        _preprocess_body,
        grid=(N_PAD // ROW_BLK,),
        in_specs=[pl.BlockSpec((ROW_BLK, D), lambda i: (i, 0))],
        out_specs=pl.BlockSpec((ROW_BLK, D), lambda i: (i, 0)),
        out_shape=jax.ShapeDtypeStruct((N_PAD, D), jnp.float32),
    )(u_pad)

    x = x0
    acc = x0
    deg_inv = None
    for li, (W, b) in enumerate(((W0, b0), (W1, b1), (W2, b2))):
        if li == 0:
            agg, deg_parts = _sc_seg_first(x, src, dst, w)
            deg_in = deg_parts
            deg_spec = pl.BlockSpec((NC, ROW_BLK), lambda i: (0, i))
        else:
            agg = _sc_seg_rest(x, src, dst, w)
            if isinstance(agg, (list, tuple)):
                agg = agg[0]
            deg_in = deg_inv
            deg_spec = pl.BlockSpec((ROW_BLK, 1), lambda i: (i, 0))
        x, acc, dinv = pl.pallas_call(
            functools.partial(_layer_body, first=(li == 0), final=(li == 2)),
            grid=(N_PAD // ROW_BLK,),
            in_specs=[
                pl.BlockSpec((NC, ROW_BLK, D), lambda i: (0, i, 0)),
                deg_spec,
                pl.BlockSpec((ROW_BLK, D), lambda i: (i, 0)),
                pl.BlockSpec((D, D), lambda i: (0, 0)),
                pl.BlockSpec((1, D), lambda i: (0, 0)),
                pl.BlockSpec((ROW_BLK, D), lambda i: (i, 0)),
            ],
            out_specs=[
                pl.BlockSpec((ROW_BLK, D), lambda i: (i, 0)),
                pl.BlockSpec((ROW_BLK, D), lambda i: (i, 0)),
                pl.BlockSpec((ROW_BLK, 1), lambda i: (i, 0)),
            ],
            out_shape=[
                jax.ShapeDtypeStruct((N_PAD, D), jnp.float32),
                jax.ShapeDtypeStruct((N_PAD, D), jnp.float32),
                jax.ShapeDtypeStruct((N_PAD, 1), jnp.float32),
            ],
        )(agg, deg_in, x, W, b.reshape(1, D), acc)
        if li == 0:
            deg_inv = dinv
    return acc[:N]


# speedup vs baseline: 1.0744x; 1.0744x over previous
import dataclasses
import functools

import jax
import jax.numpy as jnp
from jax import lax
from jax.experimental import pallas as pl
from jax.experimental.pallas import tpu as pltpu
from jax.experimental.pallas import tpu_sc as plsc

N = 10000
N_PAD = 10240
D = 128
E_RAW = 320000
LANES = 16
K_CHUNK = 64
NB = 5
NSLOT = 10
GD = 3
ID = 5
NC = 2
NS = 16
N_WORKERS = NC * NS
E_PW = 10240
CHUNKS = E_PW // K_CHUNK
E_PAD = E_PW * N_WORKERS
ROWS_PS = N_PAD // NS
WB_ROWS = 128

ROW_BLK = 1024
C_CURV = 5.0
K_CURV = 1.0 / C_CURV
SQRT_K = K_CURV ** 0.5

_SC_MESH = plsc.VectorSubcoreMesh(core_axis_name="c", subcore_axis_name="s")

_SC_PARAMS = pltpu.CompilerParams()
if "needs_layout_passes" in pltpu.CompilerParams.__dataclass_fields__:
    _SC_PARAMS = dataclasses.replace(_SC_PARAMS, needs_layout_passes=False)


def _make_sc_segment(compute_deg):
    out_type = [jax.ShapeDtypeStruct((NC, N_PAD, D), jnp.float32)]
    scratch = (
        [pltpu.VMEM((K_CHUNK,), jnp.int32) for _ in range(NSLOT)]
        + [pltpu.VMEM((K_CHUNK,), jnp.int32) for _ in range(NSLOT)]
        + [pltpu.VMEM((K_CHUNK,), jnp.float32) for _ in range(NSLOT)]
        + [pltpu.VMEM((K_CHUNK, D), jnp.float32) for _ in range(NB)]
        + [pltpu.VMEM_SHARED((N_PAD, D), jnp.float32)]
        + [pltpu.SemaphoreType.DMA for _ in range(NSLOT)]
        + [pltpu.SemaphoreType.DMA for _ in range(NSLOT)]
        + [pltpu.SemaphoreType.DMA for _ in range(NB)]
        + [pltpu.SemaphoreType.DMA for _ in range(NB)]
    )
    if compute_deg:
        out_type.append(jax.ShapeDtypeStruct((NC, N_PAD), jnp.float32))
        scratch.append(pltpu.VMEM_SHARED((N_PAD,), jnp.float32))
        scratch.append(pltpu.VMEM((N_PAD // NS,), jnp.float32))

    @functools.partial(pl.kernel, out_type=out_type, mesh=_SC_MESH,
                       scratch_types=scratch, compiler_params=_SC_PARAMS)
    def sc_seg(x_hbm, src_hbm, dst_hbm, w_hbm, agg_hbm, *rest):
        if compute_deg:
            deg_hbm = rest[0]
            rest = rest[1:]
        it = iter(rest)
        srcv = [next(it) for _ in range(NSLOT)]
        dstv = [next(it) for _ in range(NSLOT)]
        wv = [next(it) for _ in range(NSLOT)]
        rows = [next(it) for _ in range(NB)]
        acc_sp = next(it)
        sem_is = [next(it) for _ in range(NSLOT)]
        sem_idw = [next(it) for _ in range(NSLOT)]
        sem_g = [next(it) for _ in range(NB)]
        sem_s = [next(it) for _ in range(NB)]
        if compute_deg:
            deg_sp = next(it)
            zdeg = next(it)
        ci = lax.axis_index("c")
        si = lax.axis_index("s")
        ebase = (ci * NS + si) * E_PW
        DEG_PS = N_PAD // NS

        def load_idx(slot, c):
            eb = ebase + c * K_CHUNK
            pltpu.async_copy(src_hbm.at[pl.ds(eb, K_CHUNK)], srcv[slot],
                             sem_is[slot])
            pltpu.async_copy(dst_hbm.at[pl.ds(eb, K_CHUNK)], dstv[slot],
                             sem_idw[slot])
            pltpu.async_copy(w_hbm.at[pl.ds(eb, K_CHUNK)], wv[slot],
                             sem_idw[slot])

        def issue_gather(slot, b):
            pltpu.make_async_copy(src_hbm.at[pl.ds(0, K_CHUNK)], srcv[slot],
                                  sem_is[slot]).wait()
            pltpu.async_copy(x_hbm.at[srcv[slot]], rows[b], sem_g[b])

        def wait_scatter(b, slot):
            pltpu.make_async_copy(rows[b], acc_sp.at[dstv[slot]],
                                  sem_s[b]).wait()
            if compute_deg:
                pltpu.make_async_copy(wv[slot], deg_sp.at[dstv[slot]],
                                      sem_s[b]).wait()

        def process(slot, b):
            pltpu.make_async_copy(x_hbm.at[srcv[slot]], rows[b],
                                  sem_g[b]).wait()
            pltpu.make_async_copy(dst_hbm.at[pl.ds(0, K_CHUNK)], dstv[slot],
                                  sem_idw[slot]).wait()
            pltpu.make_async_copy(w_hbm.at[pl.ds(0, K_CHUNK)], wv[slot],
                                  sem_idw[slot]).wait()

            @pl.loop(0, K_CHUNK, unroll=4)
            def _scale(e):
                wvec = plsc.load_gather(
                    wv[slot], [jnp.zeros((LANES,), jnp.int32) + e])
                for j in range(0, D, LANES):
                    rows[b][e, pl.ds(j, LANES)] = (
                        rows[b][e, pl.ds(j, LANES)] * wvec)

            pltpu.async_copy(rows[b], acc_sp.at[dstv[slot]], sem_s[b],
                             add=True)
            if compute_deg:
                pltpu.async_copy(wv[slot], deg_sp.at[dstv[slot]], sem_s[b],
                                 add=True)

        for c in range(ID):
            load_idx(c, c)

        zbuf = rows[NB - 1]

        @pl.loop(0, K_CHUNK)
        def _zrow(i):
            for j in range(0, D, LANES):
                zbuf[i, pl.ds(j, LANES)] = jnp.zeros((LANES,), jnp.float32)

        @pl.loop(0, ROWS_PS // K_CHUNK)
        def _zacc(k):
            rb = si * ROWS_PS + k * K_CHUNK
            pltpu.sync_copy(zbuf, acc_sp.at[pl.ds(rb, K_CHUNK)])

        if compute_deg:
            @pl.loop(0, DEG_PS, step=LANES)
            def _zdeg(i):
                zdeg[pl.ds(i, LANES)] = jnp.zeros((LANES,), jnp.float32)
            pltpu.sync_copy(zdeg, deg_sp.at[pl.ds(si * DEG_PS, DEG_PS)])

        for c in range(GD):
            issue_gather(c, c)
        plsc.subcore_barrier()

        @pl.loop(0, CHUNKS // NSLOT)
        def _ring(cc):
            base = cc * NSLOT
            for k in range(NSLOT):
                c = base + k
                b = k % NB
                bg = (k + GD) % NB
                slotg = (k + GD) % NSLOT
                slot_prev = (k + NSLOT + GD - NB) % NSLOT
                sloti = (k + ID) % NSLOT

                if k < NB - GD:
                    @pl.when(cc > 0)
                    def _(bg=bg, slot_prev=slot_prev):
                        wait_scatter(bg, slot_prev)
                else:
                    wait_scatter(bg, slot_prev)

                if k < NSLOT - GD:
                    issue_gather(slotg, bg)
                else:
                    @pl.when(cc < CHUNKS // NSLOT - 1)
                    def _(slotg=slotg, bg=bg):
                        issue_gather(slotg, bg)

                if k < NSLOT - ID:
                    load_idx(sloti, c + ID)
                else:
                    @pl.when(cc < CHUNKS // NSLOT - 1)
                    def _(sloti=sloti, c=c):
                        load_idx(sloti, c + ID)

                process(k, b)

        for c in range(CHUNKS - (NB - GD), CHUNKS):
            wait_scatter(c % NB, c % NSLOT)
        plsc.subcore_barrier()

        @pl.loop(0, ROWS_PS // K_CHUNK)
        def _wb(k):
            rb = si * ROWS_PS + k * K_CHUNK
            pltpu.sync_copy(acc_sp.at[pl.ds(rb, K_CHUNK)],
                            agg_hbm.at[ci].at[pl.ds(rb, K_CHUNK)])
        if compute_deg:
            pltpu.sync_copy(deg_sp.at[pl.ds(si * DEG_PS, DEG_PS)],
                            deg_hbm.at[ci].at[pl.ds(si * DEG_PS, DEG_PS)])

    return sc_seg


_sc_seg_first = _make_sc_segment(True)
_sc_seg_rest = _make_sc_segment(False)


def _preprocess_body(u_ref, o_ref):
    u = u_ref[...]
    col = lax.broadcasted_iota(jnp.int32, (1, D), 1)
    sp_mask = (col >= 1).astype(jnp.float32)
    usp = u * sp_mask
    s_sp = jnp.sum(usp * usp, axis=1, keepdims=True)
    xn = jnp.maximum(jnp.sqrt(s_sp), 1e-15)
    theta = xn / SQRT_K
    et = jnp.exp(theta)
    emt = jnp.exp(-theta)
    sinh_t = 0.5 * (et - emt)
    scale1 = SQRT_K * sinh_t / xn
    sp1 = scale1 * usp
    s_sp1 = jnp.sum(sp1 * sp1, axis=1, keepdims=True)
    time2 = jnp.sqrt(jnp.maximum(K_CURV + s_sp1, 1e-15))
    alpha = jnp.maximum(time2, 1.0 + 1e-7)
    sn = jnp.maximum(jnp.sqrt(s_sp1), 1e-15)
    dist = jnp.log(alpha + jnp.sqrt(alpha * alpha - 1.0))
    o_ref[...] = (dist / sn) * sp1


def _edge_w_body(d_ref, w_ref):
    d = d_ref[...]
    w_ref[...] = jnp.exp(-(d * d))


def _layer_body(agg_ref, deginv_ref, xin_ref, w_ref, b_ref, acc_ref,
                x_ref, accout_ref, deginv_out, *, first, final):
    if first:
        deg = deginv_ref[0] + deginv_ref[1] + 1.0
        deginv = (1.0 / jnp.maximum(deg, 1e-9))[:, None]
        deginv_out[...] = deginv
    else:
        deginv = deginv_ref[...]
    a = (agg_ref[0] + agg_ref[1] + xin_ref[...]) * deginv
    h = lax.dot_general(a, w_ref[...], (((1,), (0,)), ((), ())),
                        precision=lax.Precision.HIGHEST,
                        preferred_element_type=jnp.float32)
    h = h + b_ref[...]
    x = jnp.where(h > 0.0, h, 0.01 * h)
    x_ref[...] = x
    acc = acc_ref[...] + x
    if final:
        acc = acc * 0.25
    accout_ref[...] = acc


def kernel(poi_embs, edge_index, edge_attr, W0, b0, W1, b1, W2, b2):
    n = poi_embs.shape[0]
    pad = E_PAD - E_RAW
    pad_idx = jnp.arange(pad, dtype=edge_index.dtype) % n
    src = jnp.concatenate([edge_index[0], pad_idx])
    dst = jnp.concatenate([edge_index[1], pad_idx])
    dist = jnp.concatenate([
        edge_attr,
        jnp.full((pad,), 100.0, dtype=edge_attr.dtype),
    ])

    w = pl.pallas_call(
        _edge_w_body,
        out_shape=jax.ShapeDtypeStruct((E_PAD,), jnp.float32),
    )(dist)

    u_pad = jnp.concatenate(
        [poi_embs, jnp.zeros((N_PAD - N, D), jnp.float32)], axis=0)
    x0 = pl.pallas_call(
        _preprocess_body,
        grid=(N_PAD // ROW_BLK,),
        in_specs=[pl.BlockSpec((ROW_BLK, D), lambda i: (i, 0))],
        out_specs=pl.BlockSpec((ROW_BLK, D), lambda i: (i, 0)),
        out_shape=jax.ShapeDtypeStruct((N_PAD, D), jnp.float32),
    )(u_pad)

    x = x0
    acc = x0
    deg_inv = None
    for li, (W, b) in enumerate(((W0, b0), (W1, b1), (W2, b2))):
        if li == 0:
            agg, deg_parts = _sc_seg_first(x, src, dst, w)
            deg_in = deg_parts
            deg_spec = pl.BlockSpec((NC, ROW_BLK), lambda i: (0, i))
        else:
            agg = _sc_seg_rest(x, src, dst, w)
            if isinstance(agg, (list, tuple)):
                agg = agg[0]
            deg_in = deg_inv
            deg_spec = pl.BlockSpec((ROW_BLK, 1), lambda i: (i, 0))
        x, acc, dinv = pl.pallas_call(
            functools.partial(_layer_body, first=(li == 0), final=(li == 2)),
            grid=(N_PAD // ROW_BLK,),
            in_specs=[
                pl.BlockSpec((NC, ROW_BLK, D), lambda i: (0, i, 0)),
                deg_spec,
                pl.BlockSpec((ROW_BLK, D), lambda i: (i, 0)),
                pl.BlockSpec((D, D), lambda i: (0, 0)),
                pl.BlockSpec((1, D), lambda i: (0, 0)),
                pl.BlockSpec((ROW_BLK, D), lambda i: (i, 0)),
            ],
            out_specs=[
                pl.BlockSpec((ROW_BLK, D), lambda i: (i, 0)),
                pl.BlockSpec((ROW_BLK, D), lambda i: (i, 0)),
                pl.BlockSpec((ROW_BLK, 1), lambda i: (i, 0)),
            ],
            out_shape=[
                jax.ShapeDtypeStruct((N_PAD, D), jnp.float32),
                jax.ShapeDtypeStruct((N_PAD, D), jnp.float32),
                jax.ShapeDtypeStruct((N_PAD, 1), jnp.float32),
            ],
        )(agg, deg_in, x, W, b.reshape(1, D), acc)
        if li == 0:
            deg_inv = dinv
    return acc[:N]
